# baseline (speedup 1.0000x reference)
import functools

import jax
import jax.numpy as jnp
from jax import lax
from jax.experimental import pallas as pl
from jax.experimental.pallas import tpu as pltpu
from jax.experimental.pallas import tpu_sc as plsc

N = 4096
TOPK = 32
BM = 512
BK = 512

NW = 32
L = 16
CHUNKS = N // L
GROUPS = CHUNKS // L


SLICES = 4
SROWS = N // SLICES
RPW = SROWS // NW


def _make_sim_kernel(h):
    def _sim_kernel(a_ref, b_ref, out_ref, cmax_ref):
        i = pl.program_id(0)
        k = pl.program_id(1)
        nk = pl.num_programs(1)

        acc = jax.lax.dot_general(
            a_ref[...], b_ref[...],
            dimension_numbers=(((1,), (1,)), ((), ())),
            preferred_element_type=jnp.float32,
            precision=jax.lax.Precision.DEFAULT,
        )

        @pl.when(k == 0)
        def _init():
            out_ref[...] = acc

        @pl.when(k > 0)
        def _accum():
            out_ref[...] += acc

        @pl.when(k == nk - 1)
        def _finish():
            sim = jnp.maximum(out_ref[...], 0.0)
            rows = (jax.lax.broadcasted_iota(jnp.int32, (BM, N), 0)
                    + (h * (SROWS // BM) + i) * BM)
            cols = jax.lax.broadcasted_iota(jnp.int32, (BM, N), 1)
            sim = jnp.where(rows == cols, -jnp.inf, sim)
            out_ref[...] = sim
            cm = sim[:, 0:CHUNKS]
            for j in range(1, L):
                cm = jnp.maximum(cm, sim[:, j * CHUNKS:(j + 1) * CHUNKS])
            cmax_ref[...] = cm

    return _sim_kernel


def _sim(embedding, h):
    nb = SROWS // BM
    return pl.pallas_call(
        _make_sim_kernel(h),
        grid=(nb, N // BK),
        in_specs=[
            pl.BlockSpec((BM, BK), lambda i, k, h=h: (h * (SROWS // BM) + i, k)),
            pl.BlockSpec((N, BK), lambda i, k: (0, k)),
        ],
        out_specs=[
            pl.BlockSpec((BM, N), lambda i, k: (i, 0)),
            pl.BlockSpec((BM, CHUNKS), lambda i, k: (i, 0)),
        ],
        out_shape=[
            jax.ShapeDtypeStruct((SROWS, N), jnp.float32),
            jax.ShapeDtypeStruct((SROWS, CHUNKS), jnp.float32),
        ],
    )(embedding, embedding)


def _merge_chunk(sv, siv, t0k, t0i, t1k, t1i):
    rsv = lax.rev(sv, (0,))
    rsiv = lax.rev(siv, (0,))
    m = t1k > rsv
    ak = jnp.where(m, t1k, rsv)
    ai = jnp.where(m, t1i, rsiv)
    sak, sai = plsc.sort_key_val(ak, ai, descending=True)
    rsak = lax.rev(sak, (0,))
    rsai = lax.rev(sai, (0,))
    m2 = t0k > rsak
    hik = jnp.where(m2, t0k, rsak)
    hii = jnp.where(m2, t0i, rsai)
    lok = jnp.where(m2, rsak, t0k)
    loi = jnp.where(m2, rsai, t0i)
    nt0k, nt0i = plsc.sort_key_val(hik, hii, descending=True)
    nt1k, nt1i = plsc.sort_key_val(lok, loi, descending=True)
    return nt0k, nt0i, nt1k, nt1i


def _topk_body(sim, cmax, vals, idx,
               rowbuf, cmaxbuf, dirk, diri, vstage, istage, sem):
    cid = lax.axis_index("c")
    sid = lax.axis_index("s")
    wid = sid * 2 + cid
    r0 = wid * RPW
    basis = lax.iota(jnp.int32, L)

    pltpu.make_async_copy(sim.at[r0], rowbuf.at[0], sem).start()
    pltpu.make_async_copy(cmax.at[r0], cmaxbuf.at[0], sem).start()

    def row_body(r, _):
        buf = lax.rem(r, 2)
        pltpu.make_async_copy(sim.at[r0], rowbuf.at[0], sem).wait()
        pltpu.make_async_copy(cmax.at[r0], cmaxbuf.at[0], sem).wait()

        @pl.when(r + 1 < RPW)
        def _prefetch():
            nb = lax.rem(r + 1, 2)
            pltpu.make_async_copy(sim.at[r0 + r + 1], rowbuf.at[nb],
                                  sem).start()
            pltpu.make_async_copy(cmax.at[r0 + r + 1], cmaxbuf.at[nb],
                                  sem).start()

        def dir_body(g, rmax):
            cg = cmaxbuf[buf, pl.ds(g * L, L)]
            cidv = basis + g * L
            sk, si = plsc.sort_key_val(cg, cidv, descending=True)
            dirk[pl.ds(g * L, L)] = sk
            diri[pl.ds(g * L, L)] = si
            return jnp.maximum(rmax, sk)

        neg = jnp.full((L,), -jnp.inf, dtype=jnp.float32)
        rmax = lax.fori_loop(0, GROUPS, dir_body, neg)

        def visit_chunk(g, p, carry):
            t0k, t0i, t1k, t1i, tau = carry
            dk = dirk[pl.ds(g * L, L)]
            key = dk[p]

            def hit(args):
                t0k, t0i, t1k, t1i, _tau = args
                di = diri[pl.ds(g * L, L)]
                cidx = di[p]
                iv = cidx + CHUNKS * basis
                bufv = jnp.broadcast_to(buf, (L,))
                v = plsc.load_gather(rowbuf, [bufv, iv])
                sv, siv = plsc.sort_key_val(v, iv, descending=True)

                def full_merge(args):
                    sv, siv, t0k, t0i, t1k, t1i = args
                    return _merge_chunk(sv, siv, t0k, t0i, t1k, t1i)

                def shallow_merge(args):
                    sv, siv, t0k, t0i, t1k, t1i = args
                    rsv = lax.rev(sv, (0,))
                    rsiv = lax.rev(siv, (0,))
                    m = t1k > rsv
                    ak = jnp.where(m, t1k, rsv)
                    ai = jnp.where(m, t1i, rsiv)
                    nt1k, nt1i = plsc.sort_key_val(ak, ai,
                                                   descending=True)
                    return t0k, t0i, nt1k, nt1i

                t0k, t0i, t1k, t1i = lax.cond(
                    key > t0k[L - 1], full_merge, shallow_merge,
                    (sv, siv, t0k, t0i, t1k, t1i))
                return t0k, t0i, t1k, t1i, t1k[L - 1]

            def miss(args):
                return args

            return lax.cond(key > tau, hit, miss,
                            (t0k, t0i, t1k, t1i, tau))

        zero = jnp.zeros((L,), dtype=jnp.int32)
        carry = (neg, zero, neg, zero, -jnp.inf)
        for p in range(L):
            def pass_body(args):
                def g_body(g, c):
                    return visit_chunk(g, p, c)
                return lax.fori_loop(0, GROUPS, g_body, args)

            def pass_skip(args):
                return args

            carry = lax.cond(rmax[p] > carry[4], pass_body, pass_skip,
                             carry)
        t0k, t0i, t1k, t1i, _tau = carry

        vstage[r, pl.ds(0, L)] = t0k
        vstage[r, pl.ds(L, L)] = t1k
        istage[r, pl.ds(0, L)] = t0i
        istage[r, pl.ds(L, L)] = t1i
        return _

    lax.fori_loop(0, RPW, row_body, 0)

    pltpu.sync_copy(vstage, vals.at[pl.ds(r0, RPW)])
    pltpu.sync_copy(istage, idx.at[pl.ds(r0, RPW)])


def _topk_sc(sim, cmax):
    mesh = plsc.VectorSubcoreMesh(core_axis_name="c", subcore_axis_name="s")
    return pl.kernel(
        _topk_body,
        mesh=mesh,
        compiler_params=pltpu.CompilerParams(needs_layout_passes=False),
        out_type=[
            jax.ShapeDtypeStruct((SROWS, TOPK), jnp.float32),
            jax.ShapeDtypeStruct((SROWS, TOPK), jnp.int32),
        ],
        scratch_types=[
            pltpu.VMEM((2, N), jnp.float32),
            pltpu.VMEM((2, CHUNKS), jnp.float32),
            pltpu.VMEM((CHUNKS,), jnp.float32),
            pltpu.VMEM((CHUNKS,), jnp.int32),
            pltpu.VMEM((RPW, TOPK), jnp.float32),
            pltpu.VMEM((RPW, TOPK), jnp.int32),
            pltpu.SemaphoreType.DMA,
        ],
    )(sim, cmax)


def kernel(embedding):
    vparts, iparts = [], []
    for h in range(SLICES):
        sim, cmax = _sim(embedding, h)
        v, ix = _topk_sc(sim, cmax)
        vparts.append(v)
        iparts.append(ix)
    topk_vals = jnp.concatenate(vparts, axis=0)
    topk_indices = jnp.concatenate(iparts, axis=0)
    node_indices = jnp.broadcast_to(jnp.arange(N)[:, None], (N, TOPK))
    edge_index = jnp.stack(
        [node_indices.reshape(-1), topk_indices.reshape(-1)],
        axis=0,
    )
    edge_weight = topk_vals.reshape(-1)
    return (edge_index, edge_weight)


# speedup vs baseline: 1.0658x; 1.0658x over previous
import jax
import jax.numpy as jnp
from jax import lax
from jax.experimental import pallas as pl
from jax.experimental.pallas import tpu as pltpu
from jax.experimental.pallas import tpu_sc as plsc

N = 4096
TOPK = 32
BM = 512
BK = 512

NW = 32
L = 16
CHUNKS = N // L
GROUPS = CHUNKS // L


SLICES = 4
SROWS = N // SLICES
RPW = SROWS // NW


def _make_sim_kernel(h):
    def _sim_kernel(a_ref, b_ref, out_ref, cmax_ref):
        i = pl.program_id(0)
        k = pl.program_id(1)
        nk = pl.num_programs(1)

        acc = jax.lax.dot_general(
            a_ref[...], b_ref[...],
            dimension_numbers=(((1,), (1,)), ((), ())),
            preferred_element_type=jnp.float32,
            precision=jax.lax.Precision.DEFAULT,
        )

        @pl.when(k == 0)
        def _init():
            out_ref[...] = acc

        @pl.when(k > 0)
        def _accum():
            out_ref[...] += acc

        @pl.when(k == nk - 1)
        def _finish():
            sim = jnp.maximum(out_ref[...], 0.0)
            rows = (jax.lax.broadcasted_iota(jnp.int32, (BM, N), 0)
                    + (h * (SROWS // BM) + i) * BM)
            cols = jax.lax.broadcasted_iota(jnp.int32, (BM, N), 1)
            sim = jnp.where(rows == cols, -jnp.inf, sim)
            out_ref[...] = sim
            cm = sim[:, 0:CHUNKS]
            for j in range(1, L):
                cm = jnp.maximum(cm, sim[:, j * CHUNKS:(j + 1) * CHUNKS])
            cmax_ref[...] = cm

    return _sim_kernel


def _sim(embedding, h):
    nb = SROWS // BM
    return pl.pallas_call(
        _make_sim_kernel(h),
        grid=(nb, N // BK),
        in_specs=[
            pl.BlockSpec((BM, BK), lambda i, k, h=h: (h * (SROWS // BM) + i, k)),
            pl.BlockSpec((N, BK), lambda i, k: (0, k)),
        ],
        out_specs=[
            pl.BlockSpec((BM, N), lambda i, k: (i, 0)),
            pl.BlockSpec((BM, CHUNKS), lambda i, k: (i, 0)),
        ],
        out_shape=[
            jax.ShapeDtypeStruct((SROWS, N), jnp.float32),
            jax.ShapeDtypeStruct((SROWS, CHUNKS), jnp.float32),
        ],
    )(embedding, embedding)


def _merge_chunk(sv, siv, t0k, t0i, t1k, t1i):
    rsv = lax.rev(sv, (0,))
    rsiv = lax.rev(siv, (0,))
    m = t1k > rsv
    ak = jnp.where(m, t1k, rsv)
    ai = jnp.where(m, t1i, rsiv)
    sak, sai = plsc.sort_key_val(ak, ai, descending=True)
    rsak = lax.rev(sak, (0,))
    rsai = lax.rev(sai, (0,))
    m2 = t0k > rsak
    hik = jnp.where(m2, t0k, rsak)
    hii = jnp.where(m2, t0i, rsai)
    lok = jnp.where(m2, rsak, t0k)
    loi = jnp.where(m2, rsai, t0i)
    nt0k, nt0i = plsc.sort_key_val(hik, hii, descending=True)
    nt1k, nt1i = plsc.sort_key_val(lok, loi, descending=True)
    return nt0k, nt0i, nt1k, nt1i


def _topk_body(sim, cmax, vals, idx,
               rowbuf, cmaxbuf, dirk, diri, vstage, istage, sem):
    cid = lax.axis_index("c")
    sid = lax.axis_index("s")
    wid = sid * 2 + cid
    r0 = wid * RPW
    basis = lax.iota(jnp.int32, L)

    pltpu.make_async_copy(sim.at[r0], rowbuf.at[0], sem).start()
    pltpu.make_async_copy(cmax.at[r0], cmaxbuf.at[0], sem).start()

    def row_body(r, _):
        buf = lax.rem(r, 2)
        pltpu.make_async_copy(sim.at[r0], rowbuf.at[0], sem).wait()
        pltpu.make_async_copy(cmax.at[r0], cmaxbuf.at[0], sem).wait()

        @pl.when(r + 1 < RPW)
        def _prefetch():
            nb = lax.rem(r + 1, 2)
            pltpu.make_async_copy(sim.at[r0 + r + 1], rowbuf.at[nb],
                                  sem).start()
            pltpu.make_async_copy(cmax.at[r0 + r + 1], cmaxbuf.at[nb],
                                  sem).start()

        def dir_body(g, rmax):
            cg = cmaxbuf[buf, pl.ds(g * L, L)]
            cidv = basis + g * L
            sk, si = plsc.sort_key_val(cg, cidv, descending=True)
            dirk[pl.ds(g * L, L)] = sk
            diri[pl.ds(g * L, L)] = si
            return jnp.maximum(rmax, sk)

        neg = jnp.full((L,), -jnp.inf, dtype=jnp.float32)
        rmax = lax.fori_loop(0, GROUPS, dir_body, neg)

        def visit_chunk(g, p, carry):
            t0k, t0i, t1k, t1i, tau = carry
            dk = dirk[pl.ds(g * L, L)]
            key = dk[p]

            def hit(args):
                t0k, t0i, t1k, t1i, _tau = args
                di = diri[pl.ds(g * L, L)]
                cidx = di[p]
                iv = cidx + CHUNKS * basis
                bufv = jnp.broadcast_to(buf, (L,))
                v = plsc.load_gather(rowbuf, [bufv, iv])
                sv, siv = plsc.sort_key_val(v, iv, descending=True)
                t0k, t0i, t1k, t1i = _merge_chunk(sv, siv,
                                                  t0k, t0i, t1k, t1i)
                return t0k, t0i, t1k, t1i, t1k[L - 1]

            def miss(args):
                return args

            return lax.cond(key > tau, hit, miss,
                            (t0k, t0i, t1k, t1i, tau))

        zero = jnp.zeros((L,), dtype=jnp.int32)
        carry = (neg, zero, neg, zero, -jnp.inf)
        for p in range(L):
            def pass_body(args):
                def g_body(g, c):
                    return visit_chunk(g, p, c)
                return lax.fori_loop(0, GROUPS, g_body, args)

            def pass_skip(args):
                return args

            carry = lax.cond(rmax[p] > carry[4], pass_body, pass_skip,
                             carry)
        t0k, t0i, t1k, t1i, _tau = carry

        vstage[r, pl.ds(0, L)] = t0k
        vstage[r, pl.ds(L, L)] = t1k
        istage[r, pl.ds(0, L)] = t0i
        istage[r, pl.ds(L, L)] = t1i
        return _

    lax.fori_loop(0, RPW, row_body, 0)

    pltpu.sync_copy(vstage, vals.at[pl.ds(r0, RPW)])
    pltpu.sync_copy(istage, idx.at[pl.ds(r0, RPW)])


def _topk_sc(sim, cmax):
    mesh = plsc.VectorSubcoreMesh(core_axis_name="c", subcore_axis_name="s")
    return pl.kernel(
        _topk_body,
        mesh=mesh,
        compiler_params=pltpu.CompilerParams(needs_layout_passes=False),
        out_type=[
            jax.ShapeDtypeStruct((SROWS, TOPK), jnp.float32),
            jax.ShapeDtypeStruct((SROWS, TOPK), jnp.int32),
        ],
        scratch_types=[
            pltpu.VMEM((2, N), jnp.float32),
            pltpu.VMEM((2, CHUNKS), jnp.float32),
            pltpu.VMEM((CHUNKS,), jnp.float32),
            pltpu.VMEM((CHUNKS,), jnp.int32),
            pltpu.VMEM((RPW, TOPK), jnp.float32),
            pltpu.VMEM((RPW, TOPK), jnp.int32),
            pltpu.SemaphoreType.DMA,
        ],
    )(sim, cmax)


def kernel(embedding):
    vparts, iparts = [], []
    for h in range(SLICES):
        sim, cmax = _sim(embedding, h)
        v, ix = _topk_sc(sim, cmax)
        vparts.append(v)
        iparts.append(ix)
    topk_vals = jnp.concatenate(vparts, axis=0)
    topk_indices = jnp.concatenate(iparts, axis=0)
    node_indices = jnp.broadcast_to(jnp.arange(N)[:, None], (N, TOPK))
    edge_index = jnp.stack(
        [node_indices.reshape(-1), topk_indices.reshape(-1)],
        axis=0,
    )
    edge_weight = topk_vals.reshape(-1)
    return (edge_index, edge_weight)
